# baseline (speedup 1.0000x reference)
import jax
import jax.numpy as jnp
from jax import lax
from jax.experimental import pallas as pl
from jax.experimental.pallas import tpu as pltpu
from jax.experimental.pallas import tpu_sc as plsc

BATCH = 16384
EMBED_DIM = 64
NROWS = 1000000
NC = 2
NS = 16
NW = NC * NS
BPW = BATCH // NW
CCOLS = 384
CPW = 81
NXTRA = 12
TAIL0 = 2604 * CCOLS
WLCAP = 1024
NBUF = 4


def _sweep_body(uid_hbm, iid_hbm, euT_hbm, eiT_hbm, ru_hbm, ri_hbm,
                ids_v, wl_id, wl_pos, cl_id, cl_pos, chbuf, tailbuf,
                rowtmp, csem, rsem):
    wid = lax.axis_index("s") * NC + lax.axis_index("c")
    lanes = lax.iota(jnp.int32, 16)
    lo = (wid * CPW + jnp.minimum(wid, NXTRA)) * CCOLS
    nch = CPW + jnp.where(wid < NXTRA, 1, 0)
    hi = jnp.where(wid == NW - 1, NROWS, lo + nch * CCOLS)

    def one_table(ids_hbm, tab_hbm, rows_hbm):
        pltpu.sync_copy(ids_hbm, ids_v)

        def scan(g, cnt):
            v = ids_v[pl.ds(g * 16, 16)]
            pos = g * 16 + lanes
            m = jnp.logical_and(v >= lo, v < hi)
            plsc.store_compressed(wl_id.at[pl.ds(cnt, 16)], v, mask=m)
            plsc.store_compressed(wl_pos.at[pl.ds(cnt, 16)], pos, mask=m)
            pc = plsc.all_reduce_population_count(m)
            return jnp.minimum(cnt + pc[0], WLCAP)

        cnt = lax.fori_loop(0, BATCH // 16, scan, jnp.int32(0))
        nk = (cnt + 15) // 16

        def extract_group(e, ccnt, buf, gather_fn):
            rem = ccnt - e * 16
            lc = cl_id[pl.ds(e * 16, 16)]
            pp = cl_pos[pl.ds(e * 16, 16)]
            for l in range(16):

                @pl.when(l < rem)
                def _():
                    lcv = jnp.broadcast_to(lc[l], (16,))
                    for k in range(4):
                        dvec = k * 16 + lanes
                        rowtmp[l, pl.ds(k * 16, 16)] = gather_fn(dvec, lcv)
                    pltpu.async_copy(rowtmp.at[l], rows_hbm.at[pp[l]], rsem)

            for l in range(16):

                @pl.when(l < rem)
                def _():
                    pltpu.make_async_copy(
                        rowtmp.at[l], rows_hbm.at[0], rsem).wait()
            return ccnt

        def chunk_rescan(base, width):
            def rescan(k, ccnt):
                wv = wl_id[pl.ds(k * 16, 16)]
                wp = wl_pos[pl.ds(k * 16, 16)]
                m = jnp.logical_and(
                    jnp.logical_and(wv >= base, wv < base + width),
                    k * 16 + lanes < cnt)
                plsc.store_compressed(
                    cl_id.at[pl.ds(ccnt, 16)], wv - base, mask=m)
                plsc.store_compressed(
                    cl_pos.at[pl.ds(ccnt, 16)], wp, mask=m)
                pc = plsc.all_reduce_population_count(m)
                return ccnt + pc[0]

            return lax.fori_loop(0, nk, rescan, jnp.int32(0))

        for pre in range(NBUF - 1):

            @pl.when(pre < nch)
            def _():
                pltpu.async_copy(
                    tab_hbm.at[:, pl.ds(lo + pre * CCOLS, CCOLS)],
                    chbuf.at[pre], csem)

        def chunk(c, carry):
            par = lax.rem(c, NBUF)
            base = lo + c * CCOLS

            @pl.when(c + NBUF - 1 < nch)
            def _():
                pltpu.async_copy(
                    tab_hbm.at[:, pl.ds(base + (NBUF - 1) * CCOLS, CCOLS)],
                    chbuf.at[lax.rem(c + NBUF - 1, NBUF)], csem)

            pltpu.make_async_copy(
                tab_hbm.at[:, pl.ds(0, CCOLS)], chbuf.at[par], csem).wait()

            ccnt = chunk_rescan(base, CCOLS)
            parv = jnp.broadcast_to(par, (16,))

            def gather_fn(dvec, lcv):
                return plsc.load_gather(chbuf, [parv, dvec, lcv])

            lax.fori_loop(
                0, (ccnt + 15) // 16,
                lambda e, a: extract_group(e, ccnt, chbuf, gather_fn), ccnt)
            return carry

        lax.fori_loop(0, nch, chunk, 0)

        @pl.when(wid == NW - 1)
        def _():
            pltpu.sync_copy(tab_hbm.at[:, pl.ds(TAIL0, NROWS - TAIL0)],
                            tailbuf)
            ccnt = chunk_rescan(TAIL0, NROWS - TAIL0)

            def gather_fn(dvec, lcv):
                return plsc.load_gather(tailbuf, [dvec, lcv])

            lax.fori_loop(
                0, (ccnt + 15) // 16,
                lambda e, a: extract_group(e, ccnt, tailbuf, gather_fn), ccnt)

    one_table(uid_hbm, euT_hbm, ru_hbm)
    one_table(iid_hbm, eiT_hbm, ri_hbm)


def _dot_body(ru_hbm, ri_hbm, out_hbm, bu, bi, ov):
    wid = lax.axis_index("s") * NC + lax.axis_index("c")
    lanes = lax.iota(jnp.int32, 16)
    pltpu.sync_copy(ru_hbm.at[wid], bu)
    pltpu.sync_copy(ri_hbm.at[wid], bi)

    def group(g, carry):
        bvec = (g * 16 + lanes) * EMBED_DIM
        acc = jnp.zeros((16,), jnp.float32)
        for d0 in range(EMBED_DIM):
            idx = bvec + jnp.bitwise_and(d0 + lanes, EMBED_DIM - 1)
            acc = acc + plsc.load_gather(bu, [idx]) * plsc.load_gather(
                bi, [idx])
        ov[pl.ds(g * 16, 16)] = acc
        return carry

    lax.fori_loop(0, BPW // 16, group, 0)
    pltpu.sync_copy(ov, out_hbm.at[pl.ds(wid * BPW, BPW)])


@jax.jit
def kernel(x, embed_user, embed_item):
    uid = x[:, 0].astype(jnp.int32)
    iid = x[:, 1].astype(jnp.int32)
    euT = embed_user.T
    eiT = embed_item.T

    mesh = plsc.VectorSubcoreMesh(core_axis_name="c", subcore_axis_name="s")
    params = pltpu.CompilerParams(needs_layout_passes=False)

    sweep = pl.kernel(
        _sweep_body,
        out_type=(
            jax.ShapeDtypeStruct((BATCH, EMBED_DIM), jnp.float32),
            jax.ShapeDtypeStruct((BATCH, EMBED_DIM), jnp.float32),
        ),
        mesh=mesh,
        compiler_params=params,
        scratch_types=[
            pltpu.VMEM((BATCH,), jnp.int32),
            pltpu.VMEM((WLCAP + 16,), jnp.int32),
            pltpu.VMEM((WLCAP + 16,), jnp.int32),
            pltpu.VMEM((WLCAP + 16,), jnp.int32),
            pltpu.VMEM((WLCAP + 16,), jnp.int32),
            pltpu.VMEM((NBUF, EMBED_DIM, CCOLS), jnp.float32),
            pltpu.VMEM((EMBED_DIM, NROWS - TAIL0), jnp.float32),
            pltpu.VMEM((16, EMBED_DIM), jnp.float32),
            pltpu.SemaphoreType.DMA,
            pltpu.SemaphoreType.DMA,
        ],
    )
    ru, ri = sweep(uid, iid, euT, eiT)

    dot = pl.kernel(
        _dot_body,
        out_type=jax.ShapeDtypeStruct((BATCH,), jnp.float32),
        mesh=mesh,
        compiler_params=params,
        scratch_types=[
            pltpu.VMEM((BPW * EMBED_DIM,), jnp.float32),
            pltpu.VMEM((BPW * EMBED_DIM,), jnp.float32),
            pltpu.VMEM((BPW,), jnp.float32),
        ],
    )
    return dot(ru.reshape(NW, BPW * EMBED_DIM), ri.reshape(NW, BPW * EMBED_DIM))


# speedup vs baseline: 1.0218x; 1.0218x over previous
import jax
import jax.numpy as jnp
from jax import lax
from jax.experimental import pallas as pl
from jax.experimental.pallas import tpu as pltpu
from jax.experimental.pallas import tpu_sc as plsc

BATCH = 16384
EMBED_DIM = 64
NROWS = 1000000
NC = 2
NS = 16
NW = NC * NS
BPW = BATCH // NW
CCOLS = 512
CPW = 61
NXTRA = 1
TAIL0 = 1953 * CCOLS
WLCAP = 1024
NBUF = 3


def _sweep_body(uid_hbm, iid_hbm, euT_hbm, eiT_hbm, ru_hbm, ri_hbm,
                ids_v, wl_id, wl_pos, cl_id, cl_pos, chbuf, tailbuf,
                rowtmp, csem, rsem):
    wid = lax.axis_index("s") * NC + lax.axis_index("c")
    lanes = lax.iota(jnp.int32, 16)
    lo = (wid * CPW + jnp.minimum(wid, NXTRA)) * CCOLS
    nch = CPW + jnp.where(wid < NXTRA, 1, 0)
    hi = jnp.where(wid == NW - 1, NROWS, lo + nch * CCOLS)

    def one_table(ids_hbm, tab_hbm, rows_hbm):
        for pre in range(NBUF - 1):

            @pl.when(pre < nch)
            def _():
                pltpu.async_copy(
                    tab_hbm.at[:, pl.ds(lo + pre * CCOLS, CCOLS)],
                    chbuf.at[pre], csem)

        pltpu.sync_copy(ids_hbm, ids_v)
        width = hi - lo

        def scan(g, cnt):
            v = ids_v[pl.ds(g * 16, 16)]
            pos = g * 16 + lanes
            m = (v - lo).astype(jnp.uint32) < width.astype(jnp.uint32)
            plsc.store_compressed(wl_id.at[pl.ds(cnt, 16)], v, mask=m)
            plsc.store_compressed(wl_pos.at[pl.ds(cnt, 16)], pos, mask=m)
            pc = plsc.all_reduce_population_count(m)
            return jnp.minimum(cnt + pc[0], WLCAP)

        cnt = lax.fori_loop(0, BATCH // 16, scan, jnp.int32(0))
        nk = (cnt + 15) // 16

        def extract_group(e, ccnt, buf, gather_fn):
            rem = ccnt - e * 16
            lc = cl_id[pl.ds(e * 16, 16)]
            pp = cl_pos[pl.ds(e * 16, 16)]
            for l in range(16):

                @pl.when(l < rem)
                def _():
                    lcv = jnp.broadcast_to(lc[l], (16,))
                    for k in range(4):
                        dvec = k * 16 + lanes
                        rowtmp[l, pl.ds(k * 16, 16)] = gather_fn(dvec, lcv)
                    pltpu.async_copy(rowtmp.at[l], rows_hbm.at[pp[l]], rsem)

            for l in range(16):

                @pl.when(l < rem)
                def _():
                    pltpu.make_async_copy(
                        rowtmp.at[l], rows_hbm.at[0], rsem).wait()
            return ccnt

        def chunk_rescan(base, width):
            def rescan(k, ccnt):
                wv = wl_id[pl.ds(k * 16, 16)]
                wp = wl_pos[pl.ds(k * 16, 16)]
                m = jnp.logical_and(
                    jnp.logical_and(wv >= base, wv < base + width),
                    k * 16 + lanes < cnt)
                plsc.store_compressed(
                    cl_id.at[pl.ds(ccnt, 16)], wv - base, mask=m)
                plsc.store_compressed(
                    cl_pos.at[pl.ds(ccnt, 16)], wp, mask=m)
                pc = plsc.all_reduce_population_count(m)
                return ccnt + pc[0]

            return lax.fori_loop(0, nk, rescan, jnp.int32(0))

        def chunk(c, carry):
            par = lax.rem(c, NBUF)
            base = lo + c * CCOLS

            @pl.when(c + NBUF - 1 < nch)
            def _():
                pltpu.async_copy(
                    tab_hbm.at[:, pl.ds(base + (NBUF - 1) * CCOLS, CCOLS)],
                    chbuf.at[lax.rem(c + NBUF - 1, NBUF)], csem)

            pltpu.make_async_copy(
                tab_hbm.at[:, pl.ds(0, CCOLS)], chbuf.at[par], csem).wait()

            ccnt = chunk_rescan(base, CCOLS)
            parv = jnp.broadcast_to(par, (16,))

            def gather_fn(dvec, lcv):
                return plsc.load_gather(chbuf, [parv, dvec, lcv])

            lax.fori_loop(
                0, (ccnt + 15) // 16,
                lambda e, a: extract_group(e, ccnt, chbuf, gather_fn), ccnt)
            return carry

        lax.fori_loop(0, nch, chunk, 0)

        @pl.when(wid == NW - 1)
        def _():
            pltpu.sync_copy(tab_hbm.at[:, pl.ds(TAIL0, NROWS - TAIL0)],
                            tailbuf)
            ccnt = chunk_rescan(TAIL0, NROWS - TAIL0)

            def gather_fn(dvec, lcv):
                return plsc.load_gather(tailbuf, [dvec, lcv])

            lax.fori_loop(
                0, (ccnt + 15) // 16,
                lambda e, a: extract_group(e, ccnt, tailbuf, gather_fn), ccnt)

    one_table(uid_hbm, euT_hbm, ru_hbm)
    one_table(iid_hbm, eiT_hbm, ri_hbm)


def _dot_body(ru_hbm, ri_hbm, out_hbm, bu, bi, ov):
    wid = lax.axis_index("s") * NC + lax.axis_index("c")
    lanes = lax.iota(jnp.int32, 16)
    pltpu.sync_copy(ru_hbm.at[wid], bu)
    pltpu.sync_copy(ri_hbm.at[wid], bi)

    def group(g, carry):
        bvec = (g * 16 + lanes) * EMBED_DIM
        acc = jnp.zeros((16,), jnp.float32)
        for d0 in range(EMBED_DIM):
            idx = bvec + jnp.bitwise_and(d0 + lanes, EMBED_DIM - 1)
            acc = acc + plsc.load_gather(bu, [idx]) * plsc.load_gather(
                bi, [idx])
        ov[pl.ds(g * 16, 16)] = acc
        return carry

    lax.fori_loop(0, BPW // 16, group, 0)
    pltpu.sync_copy(ov, out_hbm.at[pl.ds(wid * BPW, BPW)])


@jax.jit
def kernel(x, embed_user, embed_item):
    uid = x[:, 0].astype(jnp.int32)
    iid = x[:, 1].astype(jnp.int32)
    euT = embed_user.T
    eiT = embed_item.T

    mesh = plsc.VectorSubcoreMesh(core_axis_name="c", subcore_axis_name="s")
    params = pltpu.CompilerParams(needs_layout_passes=False)

    sweep = pl.kernel(
        _sweep_body,
        out_type=(
            jax.ShapeDtypeStruct((BATCH, EMBED_DIM), jnp.float32),
            jax.ShapeDtypeStruct((BATCH, EMBED_DIM), jnp.float32),
        ),
        mesh=mesh,
        compiler_params=params,
        scratch_types=[
            pltpu.VMEM((BATCH,), jnp.int32),
            pltpu.VMEM((WLCAP + 16,), jnp.int32),
            pltpu.VMEM((WLCAP + 16,), jnp.int32),
            pltpu.VMEM((WLCAP + 16,), jnp.int32),
            pltpu.VMEM((WLCAP + 16,), jnp.int32),
            pltpu.VMEM((NBUF, EMBED_DIM, CCOLS), jnp.float32),
            pltpu.VMEM((EMBED_DIM, NROWS - TAIL0), jnp.float32),
            pltpu.VMEM((16, EMBED_DIM), jnp.float32),
            pltpu.SemaphoreType.DMA,
            pltpu.SemaphoreType.DMA,
        ],
    )
    ru, ri = sweep(uid, iid, euT, eiT)

    dot = pl.kernel(
        _dot_body,
        out_type=jax.ShapeDtypeStruct((BATCH,), jnp.float32),
        mesh=mesh,
        compiler_params=params,
        scratch_types=[
            pltpu.VMEM((BPW * EMBED_DIM,), jnp.float32),
            pltpu.VMEM((BPW * EMBED_DIM,), jnp.float32),
            pltpu.VMEM((BPW,), jnp.float32),
        ],
    )
    return dot(ru.reshape(NW, BPW * EMBED_DIM), ri.reshape(NW, BPW * EMBED_DIM))


# speedup vs baseline: 1.0915x; 1.0682x over previous
import jax
import jax.numpy as jnp
from jax import lax
from jax.experimental import pallas as pl
from jax.experimental.pallas import tpu as pltpu
from jax.experimental.pallas import tpu_sc as plsc

BATCH = 16384
EMBED_DIM = 64
NROWS = 1000000
NC = 2
NS = 16
NW = NC * NS
BPW = BATCH // NW
CCOLS = 512
CPW = 61
NXTRA = 1
TAIL0 = 1953 * CCOLS
WLCAP = 1024
NBUF = 3


def _sweep_body(uid_hbm, iid_hbm, euT_hbm, eiT_hbm, ru_hbm, ri_hbm,
                ids_v, wl_id, wl_pos, cl_id, cl_pos, chbuf, tailbuf,
                rowtmp, csem, rsem):
    wid = lax.axis_index("s") * NC + lax.axis_index("c")
    lanes = lax.iota(jnp.int32, 16)
    lo = (wid * CPW + jnp.minimum(wid, NXTRA)) * CCOLS
    nch = CPW + jnp.where(wid < NXTRA, 1, 0)
    hi = jnp.where(wid == NW - 1, NROWS, lo + nch * CCOLS)

    def one_table(ids_hbm, tab_hbm, rows_hbm):
        for pre in range(NBUF - 1):

            @pl.when(pre < nch)
            def _():
                pltpu.async_copy(
                    tab_hbm.at[:, pl.ds(lo + pre * CCOLS, CCOLS)],
                    chbuf.at[pre], csem)

        pltpu.sync_copy(ids_hbm, ids_v)
        width = hi - lo

        def scan(g, cnt):
            v = ids_v[pl.ds(g * 16, 16)]
            pos = g * 16 + lanes
            m = (v - lo).astype(jnp.uint32) < width.astype(jnp.uint32)
            plsc.store_compressed(wl_id.at[pl.ds(cnt, 16)], v, mask=m)
            plsc.store_compressed(wl_pos.at[pl.ds(cnt, 16)], pos, mask=m)
            pc = plsc.all_reduce_population_count(m)
            return jnp.minimum(cnt + pc[0], WLCAP)

        cnt = lax.fori_loop(0, BATCH // 16, scan, jnp.int32(0))
        nk = (cnt + 15) // 16

        def extract_group(e, ccnt, buf, gather_fn):
            rem = ccnt - e * 16
            lc = cl_id[pl.ds(e * 16, 16)]
            pp = cl_pos[pl.ds(e * 16, 16)]
            for l in range(16):

                @pl.when(l < rem)
                def _():
                    lcv = jnp.broadcast_to(lc[l], (16,))
                    for k in range(4):
                        dvec = k * 16 + lanes
                        rowtmp[l, pl.ds(k * 16, 16)] = gather_fn(dvec, lcv)
                    pltpu.async_copy(
                        rowtmp.at[l],
                        rows_hbm.at[pl.ds(pp[l] * EMBED_DIM, EMBED_DIM)],
                        rsem)

            for l in range(16):

                @pl.when(l < rem)
                def _():
                    pltpu.make_async_copy(
                        rowtmp.at[l],
                        rows_hbm.at[pl.ds(0, EMBED_DIM)], rsem).wait()
            return ccnt

        def chunk_rescan(base, width):
            def rescan(k, ccnt):
                wv = wl_id[pl.ds(k * 16, 16)]
                wp = wl_pos[pl.ds(k * 16, 16)]
                m = jnp.logical_and(
                    jnp.logical_and(wv >= base, wv < base + width),
                    k * 16 + lanes < cnt)
                plsc.store_compressed(
                    cl_id.at[pl.ds(ccnt, 16)], wv - base, mask=m)
                plsc.store_compressed(
                    cl_pos.at[pl.ds(ccnt, 16)], wp, mask=m)
                pc = plsc.all_reduce_population_count(m)
                return ccnt + pc[0]

            return lax.fori_loop(0, nk, rescan, jnp.int32(0))

        def chunk(c, carry):
            par = lax.rem(c, NBUF)
            base = lo + c * CCOLS

            @pl.when(c + NBUF - 1 < nch)
            def _():
                pltpu.async_copy(
                    tab_hbm.at[:, pl.ds(base + (NBUF - 1) * CCOLS, CCOLS)],
                    chbuf.at[lax.rem(c + NBUF - 1, NBUF)], csem)

            pltpu.make_async_copy(
                tab_hbm.at[:, pl.ds(0, CCOLS)], chbuf.at[par], csem).wait()

            ccnt = chunk_rescan(base, CCOLS)
            parv = jnp.broadcast_to(par, (16,))

            def gather_fn(dvec, lcv):
                return plsc.load_gather(chbuf, [parv, dvec, lcv])

            lax.fori_loop(
                0, (ccnt + 15) // 16,
                lambda e, a: extract_group(e, ccnt, chbuf, gather_fn), ccnt)
            return carry

        lax.fori_loop(0, nch, chunk, 0)

        @pl.when(wid == NW - 1)
        def _():
            pltpu.sync_copy(tab_hbm.at[:, pl.ds(TAIL0, NROWS - TAIL0)],
                            tailbuf)
            ccnt = chunk_rescan(TAIL0, NROWS - TAIL0)

            def gather_fn(dvec, lcv):
                return plsc.load_gather(tailbuf, [dvec, lcv])

            lax.fori_loop(
                0, (ccnt + 15) // 16,
                lambda e, a: extract_group(e, ccnt, tailbuf, gather_fn), ccnt)

    one_table(uid_hbm, euT_hbm, ru_hbm)
    one_table(iid_hbm, eiT_hbm, ri_hbm)


def _dot_body(ru_hbm, ri_hbm, out_hbm, bu, bi, ov):
    wid = lax.axis_index("s") * NC + lax.axis_index("c")
    lanes = lax.iota(jnp.int32, 16)
    blk = BPW * EMBED_DIM
    pltpu.sync_copy(ru_hbm.at[pl.ds(wid * blk, blk)], bu)
    pltpu.sync_copy(ri_hbm.at[pl.ds(wid * blk, blk)], bi)

    def group(g, carry):
        bvec = (g * 16 + lanes) * EMBED_DIM
        acc = jnp.zeros((16,), jnp.float32)
        for d0 in range(EMBED_DIM):
            idx = bvec + jnp.bitwise_and(d0 + lanes, EMBED_DIM - 1)
            acc = acc + plsc.load_gather(bu, [idx]) * plsc.load_gather(
                bi, [idx])
        ov[pl.ds(g * 16, 16)] = acc
        return carry

    lax.fori_loop(0, BPW // 16, group, 0)
    pltpu.sync_copy(ov, out_hbm.at[pl.ds(wid * BPW, BPW)])


@jax.jit
def kernel(x, embed_user, embed_item):
    uid = x[:, 0].astype(jnp.int32)
    iid = x[:, 1].astype(jnp.int32)
    euT = embed_user.T
    eiT = embed_item.T

    mesh = plsc.VectorSubcoreMesh(core_axis_name="c", subcore_axis_name="s")
    params = pltpu.CompilerParams(needs_layout_passes=False)

    sweep = pl.kernel(
        _sweep_body,
        out_type=(
            jax.ShapeDtypeStruct((BATCH * EMBED_DIM,), jnp.float32),
            jax.ShapeDtypeStruct((BATCH * EMBED_DIM,), jnp.float32),
        ),
        mesh=mesh,
        compiler_params=params,
        scratch_types=[
            pltpu.VMEM((BATCH,), jnp.int32),
            pltpu.VMEM((WLCAP + 16,), jnp.int32),
            pltpu.VMEM((WLCAP + 16,), jnp.int32),
            pltpu.VMEM((WLCAP + 16,), jnp.int32),
            pltpu.VMEM((WLCAP + 16,), jnp.int32),
            pltpu.VMEM((NBUF, EMBED_DIM, CCOLS), jnp.float32),
            pltpu.VMEM((EMBED_DIM, NROWS - TAIL0), jnp.float32),
            pltpu.VMEM((16, EMBED_DIM), jnp.float32),
            pltpu.SemaphoreType.DMA,
            pltpu.SemaphoreType.DMA,
        ],
    )
    ru, ri = sweep(uid, iid, euT, eiT)

    dot = pl.kernel(
        _dot_body,
        out_type=jax.ShapeDtypeStruct((BATCH,), jnp.float32),
        mesh=mesh,
        compiler_params=params,
        scratch_types=[
            pltpu.VMEM((BPW * EMBED_DIM,), jnp.float32),
            pltpu.VMEM((BPW * EMBED_DIM,), jnp.float32),
            pltpu.VMEM((BPW,), jnp.float32),
        ],
    )
    return dot(ru, ri)
